# baseline (speedup 1.0000x reference)
import functools

import jax
import jax.numpy as jnp
from jax import lax
from jax.experimental import pallas as pl
from jax.experimental.pallas import tpu as pltpu
from jax.experimental.pallas import tpu_sc as plsc

_EPS = 1e-6
_NC = 2
_NS = 16


def kernel(x, seg, emb, posemb, segemb, gamma, beta):
    B, L = x.shape
    _, D = emb.shape
    N = B * L
    NW = _NC * _NS
    rows_per_w = N // NW
    C = 320 if rows_per_w % 320 == 0 else rows_per_w
    nchunk = rows_per_w // C
    nv = D // 16

    mesh = plsc.VectorSubcoreMesh(
        core_axis_name="c", subcore_axis_name="s",
        num_cores=_NC, num_subcores=_NS)

    @functools.partial(
        pl.kernel,
        out_type=jax.ShapeDtypeStruct((N, D), jnp.float32),
        mesh=mesh,
        compiler_params=pltpu.CompilerParams(
            needs_layout_passes=False, use_tc_tiling_on_sc=False),
        scratch_types=[
            pltpu.VMEM((C,), jnp.int32),
            pltpu.VMEM((rows_per_w + 16,), jnp.int32),
            pltpu.VMEM((L, D), jnp.float32),
            pltpu.VMEM((2, D), jnp.float32),
            pltpu.VMEM((D,), jnp.float32),
            pltpu.VMEM((D,), jnp.float32),
            pltpu.VMEM((C, D), jnp.float32),
            pltpu.VMEM((C, D), jnp.float32),
            pltpu.SemaphoreType.DMA,
        ],
    )
    def k(x_hbm, seg_hbm, emb_hbm, pos_hbm, segemb_hbm, gamma_hbm, beta_hbm,
          out_hbm, idx_v, seg_v, pos_v, segtab_v, gamma_v, beta_v, rows_v,
          out_v, sem):
        wid = lax.axis_index("s") * _NC + lax.axis_index("c")
        base = wid * rows_per_w
        pltpu.sync_copy(seg_hbm.at[pl.ds(base, rows_per_w)],
                        seg_v.at[pl.ds(0, rows_per_w)])
        pltpu.sync_copy(pos_hbm.at[pl.ds(0, L)], pos_v)
        pltpu.sync_copy(segemb_hbm, segtab_v)
        pltpu.sync_copy(gamma_hbm, gamma_v)
        pltpu.sync_copy(beta_hbm, beta_v)

        g = [gamma_v[pl.ds(16 * j, 16)] for j in range(nv)]
        bt = [beta_v[pl.ds(16 * j, 16)] for j in range(nv)]

        def chunk_body(c, _):
            row0 = pl.multiple_of(c * C, 8)
            pltpu.sync_copy(x_hbm.at[pl.ds(base + row0, C)], idx_v)
            pltpu.async_copy(emb_hbm.at[idx_v], rows_v, sem).wait()

            def row_body(r, _):
                l = lax.rem(base + row0 + r, L)
                s = seg_v[pl.ds(row0 + r, 16)][0]
                e = []
                for j in range(nv):
                    ev = rows_v[r, pl.ds(16 * j, 16)]
                    pv = pos_v[l, pl.ds(16 * j, 16)]
                    sv = segtab_v[s, pl.ds(16 * j, 16)]
                    e.append(ev + pv + sv)
                t = (e[0] + e[1]) + (e[2] + e[3])
                q = (e[0] * e[0] + e[1] * e[1]) + (e[2] * e[2] + e[3] * e[3])
                mean = jnp.sum(t) * (1.0 / D)
                var = jnp.sum(q) * (1.0 / D) - mean * mean
                vv = jnp.full((16,), var + _EPS, jnp.float32)
                iy = plsc.bitcast(vv, jnp.int32)
                y = plsc.bitcast(jnp.int32(0x5F3759DF) - (iy >> 1), jnp.float32)
                for _ in range(3):
                    y = y * (1.5 - 0.5 * vv * y * y)
                for j in range(nv):
                    out_v[r, pl.ds(16 * j, 16)] = (e[j] - mean) * y * g[j] + bt[j]
                return 0

            lax.fori_loop(0, C, row_body, 0)
            pltpu.sync_copy(out_v, out_hbm.at[pl.ds(base + row0, C)])
            return 0

        lax.fori_loop(0, nchunk, chunk_body, 0)

    out = k(x.reshape(-1), seg.reshape(-1), emb, posemb, segemb, gamma, beta)
    return out.reshape(B, L, D)
